# Initial kernel scaffold; baseline (speedup 1.0000x reference)
import jax
import jax.numpy as jnp
from jax.experimental import pallas as pl


def kernel(v, vhat, d, g, F, negatives, mask):
    raise NotImplementedError("write your pallas kernel here")


# speedup vs baseline: 8.0626x; 8.0626x over previous
import jax
import jax.numpy as jnp
from jax import lax
from jax.experimental import pallas as pl

B, D, K, N, T = 4096, 256, 512, 64, 16
BB = 512
GRID = B // BB
LAMBDA_ORTHO = 0.01


def _body(v_ref, vh_ref, g_ref, f_ref, neg_ref, mask_ref, out_ref):
    i = pl.program_id(0)
    v = v_ref[...]
    vh = vh_ref[...]
    g = g_ref[...]
    F = f_ref[...]
    neg = neg_ref[...]
    mcol = mask_ref[:, 0:1]

    base = jnp.sqrt(jnp.sum((vh - v) ** 2, axis=1, keepdims=True) + 1e-8)
    vn = jnp.sum(vh * vh, axis=1, keepdims=True)

    nn = jnp.sum(neg * neg, axis=1)
    sneg = jnp.dot(vh, neg.T, preferred_element_type=jnp.float32)
    nd = jnp.sqrt(jnp.maximum(vn - 2.0 * sneg + nn[None, :], 0.0) + 1e-8)
    ju_row = jnp.sum(jnp.maximum(1.0 + base - nd, 0.0), axis=1, keepdims=True) / N

    kiota = lax.broadcasted_iota(jnp.int32, (BB, K), 1)
    gw = g
    msel = jnp.zeros((BB, K), dtype=jnp.bool_)
    for _ in range(T):
        m = jnp.min(gw, axis=1, keepdims=True)
        eq = gw == m
        first = jnp.min(jnp.where(eq, kiota, K), axis=1, keepdims=True)
        sel = kiota == first
        msel = jnp.logical_or(msel, sel)
        gw = jnp.where(sel, jnp.inf, gw)

    sum_g = jnp.sum(jnp.where(msel, g, 0.0), axis=1, keepdims=True)
    g_t = g / (sum_g + 1e-10)
    m_t = (1.0 - g_t) ** 2

    fn = jnp.sum(F * F, axis=1)
    s = jnp.dot(vh, F.T, preferred_element_type=jnp.float32)
    dft = jnp.sqrt(jnp.maximum(vn - 2.0 * s + fn[None, :], 0.0) + 1e-8)
    hin = jnp.maximum(m_t + base - dft, 0.0)
    jt_row = jnp.sum(jnp.where(msel, hin, 0.0), axis=1, keepdims=True)

    ju_part = jnp.sum(ju_row * mcol)
    jt_part = jnp.sum(jt_row * mcol)
    mk_part = jnp.sum(mcol)

    lanes = lax.broadcasted_iota(jnp.int32, (1, 1, 128), 2)
    vals = (ju_part * (lanes == 0) + jt_part * (lanes == 1)
            + mk_part * (lanes == 2)).astype(jnp.float32)
    out_ref[...] = vals

    @pl.when(i == 0)
    def _ortho():
        gram = jnp.dot(F, F.T, preferred_element_type=jnp.float32)
        r = lax.broadcasted_iota(jnp.int32, (K, K), 0)
        c = lax.broadcasted_iota(jnp.int32, (K, K), 1)
        eye = (r == c).astype(jnp.float32)
        o = jnp.sum(jnp.abs(gram - eye))
        out_ref[...] = vals + o * (lanes == 3)


def kernel(v, vhat, d, g, F, negatives, mask):
    del d
    mask2 = jnp.broadcast_to(mask.astype(jnp.float32)[:, None], (B, 128))
    parts = pl.pallas_call(
        _body,
        grid=(GRID,),
        in_specs=[
            pl.BlockSpec((BB, D), lambda i: (i, 0)),
            pl.BlockSpec((BB, D), lambda i: (i, 0)),
            pl.BlockSpec((BB, K), lambda i: (i, 0)),
            pl.BlockSpec((K, D), lambda i: (0, 0)),
            pl.BlockSpec((N, D), lambda i: (0, 0)),
            pl.BlockSpec((BB, 128), lambda i: (i, 0)),
        ],
        out_specs=pl.BlockSpec((1, 1, 128), lambda i: (i, 0, 0)),
        out_shape=jax.ShapeDtypeStruct((GRID, 1, 128), jnp.float32),
    )(v, vhat, g, F, negatives, mask2)
    sums = jnp.sum(parts, axis=(0, 1))
    ju = sums[0] / sums[2]
    jt = sums[1] / jnp.maximum(sums[2], 1.0)
    ortho = sums[3]
    return ju + jt + LAMBDA_ORTHO * ortho ** 2
